# baseline (speedup 1.0000x reference)
import functools

import jax
import jax.numpy as jnp
from jax import lax
from jax.experimental import pallas as pl
from jax.experimental.pallas import tpu as pltpu
from jax.experimental.pallas import tpu_sc as plsc

_N = 16384
_D = 256
_K = 8192
_BETA = 0.25


_BN = 512


def _argmin_body(x_ref, cb_ref, idx_ref, cn_ref):
    i = pl.program_id(0)

    @pl.when(i == 0)
    def _():
        cb = cb_ref[...]
        cn = jnp.sum(cb * cb, axis=1)
        cn_ref[...] = cn[None, :]

    x = x_ref[...]
    cb = cb_ref[...]
    sx = jnp.sum(x * x, axis=1, keepdims=True)
    prod2 = lax.dot_general(x + x, cb, (((1,), (1,)), ((), ())))
    scores = (sx + cn_ref[...]) - prod2
    idx = jnp.argmin(scores, axis=1).astype(jnp.int32)
    idx_ref[...] = idx


def _argmin_call(x, cb):
    n = x.shape[0]
    return pl.pallas_call(
        _argmin_body,
        grid=(n // _BN,),
        in_specs=[
            pl.BlockSpec((_BN, _D), lambda i: (i, 0)),
            pl.BlockSpec((_K, _D), lambda i: (0, 0)),
        ],
        out_specs=pl.BlockSpec((_BN,), lambda i: (i,)),
        out_shape=jax.ShapeDtypeStruct((n,), jnp.int32),
        scratch_shapes=[pltpu.VMEM((1, _K), jnp.float32)],
    )(x, cb)


_NC = 2
_NS = 16
_NW = _NC * _NS
_CH = 128


def _make_gather_body(bpw):
    nchunk = bpw // _CH

    def _gather_body(table_hbm, idx_hbm, out_hbm, idx_v, buf0, buf1,
                     gs0, gs1, ws0, ws1):
        wid = lax.axis_index("s") * _NC + lax.axis_index("c")
        base = wid * bpw
        pltpu.sync_copy(idx_hbm.at[pl.ds(base, bpw)], idx_v)
        bufs = (buf0, buf1)
        gsems = (gs0, gs1)
        wsems = (ws0, ws1)

        def fire_gather(c):
            return pltpu.async_copy(
                table_hbm.at[idx_v.at[pl.ds(c * _CH, _CH)]], bufs[c % 2],
                gsems[c % 2])

        gathers = [fire_gather(0)]
        writes = []
        for c in range(nchunk):
            if c + 1 < nchunk:
                if c >= 1:
                    writes[c - 1].wait()
                gathers.append(fire_gather(c + 1))
            gathers[c].wait()
            writes.append(pltpu.async_copy(
                bufs[c % 2], out_hbm.at[pl.ds(base + c * _CH, _CH)],
                wsems[c % 2]))
        for w in writes[-2:]:
            w.wait()

    return _gather_body


def _gather_call(codebook, idx):
    n = idx.shape[0]
    bpw = n // _NW
    mesh = plsc.VectorSubcoreMesh(core_axis_name="c", subcore_axis_name="s")
    f = functools.partial(
        pl.kernel,
        mesh=mesh,
        out_type=jax.ShapeDtypeStruct((n, _D), jnp.float32),
        scratch_types=[
            pltpu.VMEM((bpw,), jnp.int32),
            pltpu.VMEM((_CH, _D), jnp.float32),
            pltpu.VMEM((_CH, _D), jnp.float32),
            pltpu.SemaphoreType.DMA,
            pltpu.SemaphoreType.DMA,
            pltpu.SemaphoreType.DMA,
            pltpu.SemaphoreType.DMA,
        ],
    )(_make_gather_body(bpw))
    return f(codebook, idx)


_BM = 512


def _loss_body(x_ref, y_ref, q_ref, qo_ref, loss_ref):
    i = pl.program_id(0)
    q = q_ref[...]
    x = x_ref[...]
    y = y_ref[...]
    t = x + y
    qo_ref[...] = t + (q - t)
    dx = q - x
    dy = q - y
    s = jnp.sum(dx * dx) + jnp.sum(dy * dy)

    @pl.when(i == 0)
    def _():
        loss_ref[0, 0] = 0.0

    loss_ref[0, 0] += s


def _loss_call(x, y, q):
    n = x.shape[0]
    return pl.pallas_call(
        _loss_body,
        grid=(n // _BM,),
        in_specs=[
            pl.BlockSpec((_BM, _D), lambda i: (i, 0)),
            pl.BlockSpec((_BM, _D), lambda i: (i, 0)),
            pl.BlockSpec((_BM, _D), lambda i: (i, 0)),
        ],
        out_specs=[
            pl.BlockSpec((_BM, _D), lambda i: (i, 0)),
            pl.BlockSpec(memory_space=pltpu.SMEM),
        ],
        out_shape=[
            jax.ShapeDtypeStruct((n, _D), jnp.float32),
            jax.ShapeDtypeStruct((1, 1), jnp.float32),
        ],
    )(x, y, q)


def kernel(x, y, codebook):
    h = _N // 2
    idx1 = _argmin_call(x[:h], codebook)
    q1 = _gather_call(codebook, idx1)
    idx2 = _argmin_call(x[h:], codebook)
    q2 = _gather_call(codebook, idx2)
    qo1, s1 = _loss_call(x[:h], y[:h], q1)
    qo2, s2 = _loss_call(x[h:], y[h:], q2)
    qo = jnp.concatenate([qo1, qo2], axis=0)
    loss = (s1[0, 0] + s2[0, 0]) * ((1.0 + _BETA) / (_N * _D))
    return qo, loss


# speedup vs baseline: 1.0865x; 1.0865x over previous
import functools

import jax
import jax.numpy as jnp
from jax import lax
from jax.experimental import pallas as pl
from jax.experimental.pallas import tpu as pltpu
from jax.experimental.pallas import tpu_sc as plsc

_N = 16384
_D = 256
_K = 8192
_BETA = 0.25


_BN = 512


def _argmin_body(x_ref, cb_ref, idx_ref, cn_ref):
    i = pl.program_id(0)

    @pl.when(i == 0)
    def _():
        cb = cb_ref[...]
        cn = jnp.sum(cb * cb, axis=1)
        cn_ref[...] = cn[None, :]

    x = x_ref[...]
    cb = cb_ref[...]
    sx = jnp.sum(x * x, axis=1, keepdims=True)
    prod2 = lax.dot_general(x + x, cb, (((1,), (1,)), ((), ())))
    scores = (sx + cn_ref[...]) - prod2
    idx = jnp.argmin(scores, axis=1).astype(jnp.int32)
    idx_ref[...] = idx


def _argmin_call(x, cb):
    n = x.shape[0]
    return pl.pallas_call(
        _argmin_body,
        grid=(n // _BN,),
        in_specs=[
            pl.BlockSpec((_BN, _D), lambda i: (i, 0)),
            pl.BlockSpec((_K, _D), lambda i: (0, 0)),
        ],
        out_specs=pl.BlockSpec((_BN,), lambda i: (i,)),
        out_shape=jax.ShapeDtypeStruct((n,), jnp.int32),
        scratch_shapes=[pltpu.VMEM((1, _K), jnp.float32)],
    )(x, cb)


_NC = 2
_NS = 16
_NW = _NC * _NS
_CH = 128


def _make_gather_body(bpw):
    nchunk = bpw // _CH

    def _gather_body(table_hbm, idx_hbm, out_hbm, idx_v, buf0, buf1,
                     gs0, gs1, ws0, ws1):
        wid = lax.axis_index("s") * _NC + lax.axis_index("c")
        base = wid * bpw
        pltpu.sync_copy(idx_hbm.at[pl.ds(base, bpw)], idx_v)
        bufs = (buf0, buf1)
        gsems = (gs0, gs1)
        wsems = (ws0, ws1)

        def fire_gather(c):
            return pltpu.async_copy(
                table_hbm.at[idx_v.at[pl.ds(c * _CH, _CH)]], bufs[c % 2],
                gsems[c % 2])

        gathers = [fire_gather(0)]
        writes = []
        for c in range(nchunk):
            if c + 1 < nchunk:
                if c >= 1:
                    writes[c - 1].wait()
                gathers.append(fire_gather(c + 1))
            gathers[c].wait()
            writes.append(pltpu.async_copy(
                bufs[c % 2], out_hbm.at[pl.ds(base + c * _CH, _CH)],
                wsems[c % 2]))
        for w in writes[-2:]:
            w.wait()

    return _gather_body


def _gather_call(codebook, idx):
    n = idx.shape[0]
    bpw = n // _NW
    mesh = plsc.VectorSubcoreMesh(core_axis_name="c", subcore_axis_name="s")
    f = functools.partial(
        pl.kernel,
        mesh=mesh,
        out_type=jax.ShapeDtypeStruct((n, _D), jnp.float32),
        scratch_types=[
            pltpu.VMEM((bpw,), jnp.int32),
            pltpu.VMEM((_CH, _D), jnp.float32),
            pltpu.VMEM((_CH, _D), jnp.float32),
            pltpu.SemaphoreType.DMA,
            pltpu.SemaphoreType.DMA,
            pltpu.SemaphoreType.DMA,
            pltpu.SemaphoreType.DMA,
        ],
    )(_make_gather_body(bpw))
    return f(codebook, idx)


_BM = 512


def _loss_body(x_ref, y_ref, q_ref, qo_ref, loss_ref):
    i = pl.program_id(0)
    q = q_ref[...]
    x = x_ref[...]
    y = y_ref[...]
    t = x + y
    qo_ref[...] = t + (q - t)
    dx = q - x
    dy = q - y
    s = jnp.sum(dx * dx) + jnp.sum(dy * dy)

    @pl.when(i == 0)
    def _():
        loss_ref[0, 0] = 0.0

    loss_ref[0, 0] += s


def _loss_call(x, y, q):
    n = x.shape[0]
    return pl.pallas_call(
        _loss_body,
        grid=(n // _BM,),
        in_specs=[
            pl.BlockSpec((_BM, _D), lambda i: (i, 0)),
            pl.BlockSpec((_BM, _D), lambda i: (i, 0)),
            pl.BlockSpec((_BM, _D), lambda i: (i, 0)),
        ],
        out_specs=[
            pl.BlockSpec((_BM, _D), lambda i: (i, 0)),
            pl.BlockSpec(memory_space=pltpu.SMEM),
        ],
        out_shape=[
            jax.ShapeDtypeStruct((n, _D), jnp.float32),
            jax.ShapeDtypeStruct((1, 1), jnp.float32),
        ],
    )(x, y, q)


def kernel(x, y, codebook):
    idx = _argmin_call(x, codebook)
    q = _gather_call(codebook, idx)
    qo, lsum = _loss_call(x, y, q)
    loss = lsum[0, 0] * ((1.0 + _BETA) / (_N * _D))
    return qo, loss


# speedup vs baseline: 1.1781x; 1.0843x over previous
import functools

import jax
import jax.numpy as jnp
from jax import lax
from jax.experimental import pallas as pl
from jax.experimental.pallas import tpu as pltpu
from jax.experimental.pallas import tpu_sc as plsc

_N = 16384
_D = 256
_K = 8192
_BETA = 0.25


_BN = 512


def _argmin_body(x_ref, y_ref, cb_ref, idx_ref, t_ref, cnout_ref, sxy_ref,
                 cn_ref):
    i = pl.program_id(0)

    @pl.when(i == 0)
    def _():
        cb = cb_ref[...]
        cn = jnp.sum(cb * cb, axis=1)
        cn_ref[...] = cn[None, :]
        sxy_ref[0, 0] = 0.0

    cnout_ref[...] = cn_ref[...]
    x = x_ref[...]
    y = y_ref[...]
    cb = cb_ref[...]
    t_ref[...] = x + y
    sx = jnp.sum(x * x, axis=1, keepdims=True)
    sxy_ref[0, 0] += jnp.sum(sx) + jnp.sum(y * y)
    prod2 = lax.dot_general(x + x, cb, (((1,), (1,)), ((), ())))
    scores = (sx + cn_ref[...]) - prod2
    idx = jnp.argmin(scores, axis=1).astype(jnp.int32)
    idx_ref[...] = idx


def _argmin_call(x, y, cb):
    n = x.shape[0]
    return pl.pallas_call(
        _argmin_body,
        grid=(n // _BN,),
        in_specs=[
            pl.BlockSpec((_BN, _D), lambda i: (i, 0)),
            pl.BlockSpec((_BN, _D), lambda i: (i, 0)),
            pl.BlockSpec((_K, _D), lambda i: (0, 0)),
        ],
        out_specs=[
            pl.BlockSpec((_BN,), lambda i: (i,)),
            pl.BlockSpec((_BN, _D), lambda i: (i, 0)),
            pl.BlockSpec((1, _K), lambda i: (0, 0)),
            pl.BlockSpec(memory_space=pltpu.SMEM),
        ],
        out_shape=[
            jax.ShapeDtypeStruct((n,), jnp.int32),
            jax.ShapeDtypeStruct((n, _D), jnp.float32),
            jax.ShapeDtypeStruct((1, _K), jnp.float32),
            jax.ShapeDtypeStruct((1, 1), jnp.float32),
        ],
        scratch_shapes=[pltpu.VMEM((1, _K), jnp.float32)],
    )(x, y, cb)


_NC = 2
_NS = 16
_NW = _NC * _NS
_CH = 64
_L = 16


def _make_fused_body(bpw):
    nchunk = bpw // _CH

    def body(table_hbm, idx_hbm, t_hbm, cn_hbm, qo_hbm, part_hbm,
             idx_v, cn_v, qb0, qb1, tb0, tb1, acc_v,
             cs, gq0, gq1, gt0, gt1, w0, w1):
        wid = lax.axis_index("s") * _NC + lax.axis_index("c")
        base = wid * bpw
        pltpu.sync_copy(idx_hbm.at[pl.ds(base, bpw)], idx_v)
        cng = pltpu.async_copy(cn_hbm.at[idx_v], cn_v, cs)
        qbs = (qb0, qb1)
        tbs = (tb0, tb1)
        gqs = (gq0, gq1)
        gts = (gt0, gt1)
        ws = (w0, w1)

        def fire(c):
            g = pltpu.async_copy(
                table_hbm.at[idx_v.at[pl.ds(c * _CH, _CH)]], qbs[c % 2],
                gqs[c % 2])
            tt = pltpu.async_copy(
                t_hbm.at[pl.ds(base + c * _CH, _CH)], tbs[c % 2], gts[c % 2])
            return g, tt

        pend = {0: fire(0)}
        writes = {}
        aqt = jnp.zeros((_L,), jnp.float32)
        for c in range(nchunk):
            if c + 1 < nchunk:
                if c >= 1:
                    writes[c - 1].wait()
                pend[c + 1] = fire(c + 1)
            g, tt = pend[c]
            g.wait()
            tt.wait()
            qb = qbs[c % 2]
            tb = tbs[c % 2]

            def row(r, a1, qb=qb, tb=tb):
                for j in range(_D // _L):
                    sl = pl.ds(j * _L, _L)
                    qv = qb[r, sl]
                    tv = tb[r, sl]
                    qb[r, sl] = tv + (qv - tv)
                    a1 = a1 + qv * tv
                return a1

            aqt = lax.fori_loop(0, _CH, row, aqt)
            writes[c] = pltpu.async_copy(
                qbs[c % 2], qo_hbm.at[pl.ds(base + c * _CH, _CH)], ws[c % 2])
        writes[nchunk - 2].wait()
        writes[nchunk - 1].wait()

        cng.wait()
        aq2 = jnp.zeros((_L,), jnp.float32)

        def cnrow(r, a2):
            return a2 + cn_v[pl.ds(r * _L, _L)]

        aq2 = lax.fori_loop(0, bpw // _L, cnrow, aq2)
        acc_v[0, pl.ds(0, _L)] = aqt
        acc_v[1, pl.ds(0, _L)] = aq2
        pltpu.sync_copy(acc_v, part_hbm.at[wid])

    return body


def _gather_call(codebook, idx, t, cn):
    n = idx.shape[0]
    bpw = n // _NW
    mesh = plsc.VectorSubcoreMesh(core_axis_name="c", subcore_axis_name="s")
    f = functools.partial(
        pl.kernel,
        mesh=mesh,
        out_type=[
            jax.ShapeDtypeStruct((n, _D), jnp.float32),
            jax.ShapeDtypeStruct((_NW, 2, _L), jnp.float32),
        ],
        scratch_types=[
            pltpu.VMEM((bpw,), jnp.int32),
            pltpu.VMEM((bpw,), jnp.float32),
            pltpu.VMEM((_CH, _D), jnp.float32),
            pltpu.VMEM((_CH, _D), jnp.float32),
            pltpu.VMEM((_CH, _D), jnp.float32),
            pltpu.VMEM((_CH, _D), jnp.float32),
            pltpu.VMEM((2, _L), jnp.float32),
            pltpu.SemaphoreType.DMA,
            pltpu.SemaphoreType.DMA,
            pltpu.SemaphoreType.DMA,
            pltpu.SemaphoreType.DMA,
            pltpu.SemaphoreType.DMA,
            pltpu.SemaphoreType.DMA,
            pltpu.SemaphoreType.DMA,
        ],
    )(_make_fused_body(bpw))
    return f(codebook, idx, t, cn)


def kernel(x, y, codebook):
    idx, t, cn, sxy = _argmin_call(x, y, codebook)
    qo, part = _gather_call(codebook, idx, t, cn.reshape(_K))
    loss_sum = (2.0 * jnp.sum(part[:, 1, :]) - 2.0 * jnp.sum(part[:, 0, :])
                + sxy[0, 0])
    loss = loss_sum * ((1.0 + _BETA) / (_N * _D))
    return qo, loss
